# baseline (speedup 1.0000x reference)
import functools
import math

import jax
import jax.numpy as jnp
from jax import lax
from jax.experimental import pallas as pl
from jax.experimental.pallas import tpu as pltpu
from jax.experimental.pallas import tpu_sc as plsc

N = 10000
E = 320000
NFEAT = 128
NHID = 128
NCLASS = 64
BN_EPS = 1e-5

NC = 2
NS = 16
CH = 128
NCHUNK = 160
EPT = NCHUNK * CH
E_PAD = NS * EPT
NB = 2
LK = 1
ACC_N = 10240
D0 = 72
D2 = 32
BR = 2048
_BN_SCALE = 1.0 / math.sqrt(1.0 + BN_EPS)


def _make_seg_sum(drow):
  mesh = plsc.VectorSubcoreMesh(core_axis_name="c", subcore_axis_name="s")
  rps = ACC_N // NS

  @functools.partial(
      pl.kernel,
      out_type=jax.ShapeDtypeStruct((NC, ACC_N, drow), jnp.float32),
      mesh=mesh,
      compiler_params=pltpu.CompilerParams(use_tc_tiling_on_sc=False),
      scratch_types=[
          pltpu.VMEM((NCHUNK, CH), jnp.int32),
          pltpu.VMEM((NCHUNK, CH), jnp.int32),
          [pltpu.VMEM((CH, drow), jnp.float32) for _ in range(NB)],
          pltpu.VMEM_SHARED((ACC_N, drow), jnp.float32),
          [pltpu.SemaphoreType.DMA for _ in range(NB)],
          [pltpu.SemaphoreType.DMA for _ in range(NB)],
      ],
  )
  def seg_sum(y_hbm, src_hbm, dst_hbm, zeros_hbm, out_hbm,
              src_v, dst_v, bufs, acc, gsem, ssem):
    c = lax.axis_index("c")
    s = lax.axis_index("s")

    pltpu.sync_copy(zeros_hbm, acc.at[pl.ds(s * rps, rps)])

    pltpu.sync_copy(src_hbm.at[c, s], src_v)
    pltpu.sync_copy(dst_hbm.at[s], dst_v)
    plsc.subcore_barrier()

    pltpu.async_copy(y_hbm.at[src_v.at[0]], bufs[0], gsem[0])
    pltpu.async_copy(y_hbm.at[src_v.at[1]], bufs[1], gsem[1])

    @pl.loop(0, (NCHUNK - 2) // 2)
    def _chunks(j):
      i = 2 * j
      pltpu.make_async_copy(y_hbm.at[src_v.at[i]], bufs[0], gsem[0]).wait()
      pltpu.sync_copy(bufs[0], acc.at[dst_v.at[i]], add=True)
      pltpu.async_copy(y_hbm.at[src_v.at[i + 2]], bufs[0], gsem[0])
      pltpu.make_async_copy(y_hbm.at[src_v.at[i + 1]], bufs[1], gsem[1]).wait()
      pltpu.sync_copy(bufs[1], acc.at[dst_v.at[i + 1]], add=True)
      pltpu.async_copy(y_hbm.at[src_v.at[i + 3]], bufs[1], gsem[1])

    pltpu.make_async_copy(y_hbm.at[src_v.at[NCHUNK - 2]], bufs[0], gsem[0]).wait()
    pltpu.sync_copy(bufs[0], acc.at[dst_v.at[NCHUNK - 2]], add=True)
    pltpu.make_async_copy(y_hbm.at[src_v.at[NCHUNK - 1]], bufs[1], gsem[1]).wait()
    pltpu.sync_copy(bufs[1], acc.at[dst_v.at[NCHUNK - 1]], add=True)

    plsc.subcore_barrier()
    pltpu.sync_copy(acc.at[pl.ds(s * rps, rps)],
                    out_hbm.at[c, pl.ds(s * rps, rps)])

  return seg_sum


_seg_sum_l0 = _make_seg_sum(D0)
_seg_sum_l2 = _make_seg_sum(D2)


def _stage_a(xp, wl0t, wr0t):
  def body(x_ref, wl_ref, wr_ref, ycat_ref, r0_ref):
    xv = x_ref[...]
    y0 = jnp.dot(xv, wl_ref[...], preferred_element_type=jnp.float32)
    ones = jnp.ones((BR, 8), jnp.float32)
    ycat_ref[0] = jnp.concatenate([y0[:, :64], ones], axis=1)
    ycat_ref[1] = jnp.concatenate([y0[:, 64:], ones], axis=1)
    r0_ref[...] = jnp.dot(xv, wr_ref[...], preferred_element_type=jnp.float32)

  return pl.pallas_call(
      body,
      grid=(ACC_N // BR,),
      in_specs=[
          pl.BlockSpec((BR, NFEAT), lambda i: (i, 0)),
          pl.BlockSpec((NFEAT, NHID), lambda i: (0, 0)),
          pl.BlockSpec((NFEAT, NHID), lambda i: (0, 0)),
      ],
      out_specs=[
          pl.BlockSpec((NC, BR, D0), lambda i: (0, i, 0)),
          pl.BlockSpec((BR, NHID), lambda i: (i, 0)),
      ],
      out_shape=[
          jax.ShapeDtypeStruct((NC, ACC_N, D0), jnp.float32),
          jax.ShapeDtypeStruct((ACC_N, NHID), jnp.float32),
      ],
  )(xp, wl0t, wr0t)


def _stage_b(p0, r0, b0, gamma, beta, wl2t, wr2t, b2):
  def body(p_ref, r0_ref, b0_ref, g_ref, be_ref, wl_ref, wr_ref, b2_ref,
           y2_ref, r2b_ref, inv8_ref):
    agg = jnp.concatenate([p_ref[0, :, :64], p_ref[1, :, :64]], axis=1)
    cnt = p_ref[0, :, 64:65]
    inv = 1.0 / jnp.maximum(cnt, 1.0)
    pre = agg * inv + r0_ref[...] + b0_ref[...]
    h = jnp.maximum(pre * (g_ref[...] * _BN_SCALE) + be_ref[...], 0.0)
    y2 = jnp.dot(h, wl_ref[...], preferred_element_type=jnp.float32)
    y2_ref[0] = y2[:, :D2]
    y2_ref[1] = y2[:, D2:]
    r2b_ref[...] = jnp.dot(h, wr_ref[...],
                           preferred_element_type=jnp.float32) + b2_ref[...]
    inv8_ref[...] = jnp.broadcast_to(inv, (BR, 8))

  return pl.pallas_call(
      body,
      grid=(ACC_N // BR,),
      in_specs=[
          pl.BlockSpec((NC, BR, D0), lambda i: (0, i, 0)),
          pl.BlockSpec((BR, NHID), lambda i: (i, 0)),
          pl.BlockSpec((NHID,), lambda i: (0,)),
          pl.BlockSpec((NHID,), lambda i: (0,)),
          pl.BlockSpec((NHID,), lambda i: (0,)),
          pl.BlockSpec((NHID, NCLASS), lambda i: (0, 0)),
          pl.BlockSpec((NHID, NCLASS), lambda i: (0, 0)),
          pl.BlockSpec((NCLASS,), lambda i: (0,)),
      ],
      out_specs=[
          pl.BlockSpec((NC, BR, D2), lambda i: (0, i, 0)),
          pl.BlockSpec((BR, NCLASS), lambda i: (i, 0)),
          pl.BlockSpec((BR, 8), lambda i: (i, 0)),
      ],
      out_shape=[
          jax.ShapeDtypeStruct((NC, ACC_N, D2), jnp.float32),
          jax.ShapeDtypeStruct((ACC_N, NCLASS), jnp.float32),
          jax.ShapeDtypeStruct((ACC_N, 8), jnp.float32),
      ],
  )(p0, r0, b0, gamma, beta, wl2t, wr2t, b2)


def _stage_c(p2, r2b, inv8):
  def body(p_ref, r_ref, inv_ref, out_ref):
    psum = jnp.concatenate([p_ref[0], p_ref[1]], axis=1)
    out_ref[...] = psum * inv_ref[:, 0:1] + r_ref[...]

  return pl.pallas_call(
      body,
      grid=(ACC_N // BR,),
      in_specs=[
          pl.BlockSpec((NC, BR, D2), lambda i: (0, i, 0)),
          pl.BlockSpec((BR, NCLASS), lambda i: (i, 0)),
          pl.BlockSpec((BR, 8), lambda i: (i, 0)),
      ],
      out_specs=pl.BlockSpec((BR, NCLASS), lambda i: (i, 0)),
      out_shape=jax.ShapeDtypeStruct((ACC_N, NCLASS), jnp.float32),
  )(p2, r2b, inv8)


@jax.jit
def kernel(x, edge_index, Wl0, Wr0, b0, gamma, beta, Wl2, Wr2, b2):
  xp = jnp.pad(x, ((0, ACC_N - N), (0, 0)))
  src = jnp.concatenate(
      [edge_index[0], jnp.zeros((E_PAD - E,), jnp.int32)]).reshape(
          NS, NCHUNK, CH)
  src2 = jnp.stack([src, src + ACC_N])
  dst2 = jnp.concatenate(
      [edge_index[1], jnp.full((E_PAD - E,), N, jnp.int32)]).reshape(
          NS, NCHUNK, CH)

  ycat, r0 = _stage_a(xp, Wl0.T, Wr0.T)
  z0 = jnp.zeros((ACC_N // NS, D0), jnp.float32)
  z2 = jnp.zeros((ACC_N // NS, D2), jnp.float32)
  p0 = _seg_sum_l0(ycat.reshape(NC * ACC_N, D0), src2, dst2, z0)
  y2, r2b, inv8 = _stage_b(p0, r0, b0, gamma, beta, Wl2.T, Wr2.T, b2)
  p2 = _seg_sum_l2(y2.reshape(NC * ACC_N, D2), src2, dst2, z2)
  out = _stage_c(p2, r2b, inv8)
  return out[:N]


# speedup vs baseline: 1.0605x; 1.0605x over previous
import functools
import math

import jax
import jax.numpy as jnp
from jax import lax
from jax.experimental import pallas as pl
from jax.experimental.pallas import tpu as pltpu
from jax.experimental.pallas import tpu_sc as plsc

N = 10000
E = 320000
NFEAT = 128
NHID = 128
NCLASS = 64
BN_EPS = 1e-5

NC = 2
NS = 16
CH = 128
NCHUNK = 160
EPT = NCHUNK * CH
E_PAD = NS * EPT
NB = 4
LK = 2
ACC_N = 10016
D0 = 64
D2 = 32
BR = 2504
_BN_SCALE = 1.0 / math.sqrt(1.0 + BN_EPS)


def _make_seg_sum(drow, deep, with_count):
  mesh = plsc.VectorSubcoreMesh(core_axis_name="c", subcore_axis_name="s")
  rps = ACC_N // NS

  outs = jax.ShapeDtypeStruct((NC, ACC_N, drow), jnp.float32)
  if with_count:
    outs = (outs, jax.ShapeDtypeStruct((NS, ACC_N), jnp.float32))

  @functools.partial(
      pl.kernel,
      out_type=outs,
      mesh=mesh,
      compiler_params=pltpu.CompilerParams(use_tc_tiling_on_sc=False, needs_layout_passes=False),
      scratch_types=[
          pltpu.VMEM((NCHUNK, CH), jnp.int32),
          pltpu.VMEM((NCHUNK, CH), jnp.int32),
          [pltpu.VMEM((CH, drow), jnp.float32) for _ in range(NB)],
          pltpu.VMEM((ACC_N,), jnp.float32),
          pltpu.VMEM_SHARED((ACC_N, drow), jnp.float32),
          [pltpu.SemaphoreType.DMA for _ in range(NB)],
          [pltpu.SemaphoreType.DMA for _ in range(NB)],
      ],
  )
  def seg_sum(y_hbm, src_hbm, dst_hbm, zeros_hbm, *rest):
    if with_count:
      (out_hbm, cnt_hbm,
       src_v, dst_v, bufs, cnt_v, acc, gsem, ssem) = rest
    else:
      cnt_hbm = None
      (out_hbm,
       src_v, dst_v, bufs, cnt_v, acc, gsem, ssem) = rest
    c = lax.axis_index("c")
    s = lax.axis_index("s")

    pltpu.sync_copy(zeros_hbm, acc.at[pl.ds(s * rps, rps)])

    pltpu.sync_copy(src_hbm.at[c, s], src_v)
    pltpu.sync_copy(dst_hbm.at[s], dst_v)

    if with_count:
      zv16 = jnp.zeros((16,), jnp.float32)
      for k in range(ACC_N // 16):
        cnt_v[pl.ds(k * 16, 16)] = zv16

    ones16 = jnp.ones((16,), jnp.float32)

    def count_chunk(i):
      if with_count:
        @pl.when(c == 0)
        def _():
          for k in range(CH // 16):
            idx = dst_v[i, pl.ds(k * 16, 16)]
            plsc.addupdate_scatter(cnt_v, [idx], ones16)

    plsc.subcore_barrier()

    def fire_gather(i, b):
      pltpu.async_copy(y_hbm.at[src_v.at[i]], bufs[b], gsem[b])

    def wait_gather(i, b):
      pltpu.make_async_copy(y_hbm.at[src_v.at[i]], bufs[b], gsem[b]).wait()

    def fire_scatter(i, b):
      pltpu.async_copy(bufs[b], acc.at[dst_v.at[i]], ssem[b], add=True)

    def wait_scatter(i, b):
      pltpu.make_async_copy(bufs[b], acc.at[dst_v.at[i]], ssem[b]).wait()

    if deep:
      for b in range(LK):
        fire_gather(b, b)

      @pl.loop(0, NCHUNK // NB)
      def _steady(j):
        for t in range(NB):
          i = NB * j + t
          b = t
          bk = (t + LK) % NB
          wait_gather(i, b)
          fire_scatter(i, b)
          count_chunk(i)
          if t < LK:
            @pl.when(j > 0)
            def _():
              wait_scatter(i - LK, bk)
          else:
            wait_scatter(i - LK, bk)

          @pl.when(i + LK < NCHUNK)
          def _():
            fire_gather(i + LK, bk)

      for t in range(LK):
        i = NCHUNK - LK + t
        wait_scatter(i, i % NB)
    else:
      for b in range(2):
        fire_gather(b, b)

      @pl.loop(0, NCHUNK // 2)
      def _steady(j):
        for t in range(2):
          i = 2 * j + t
          wait_gather(i, t)
          pltpu.sync_copy(bufs[t], acc.at[dst_v.at[i]], add=True)
          count_chunk(i)

          @pl.when(i + 2 < NCHUNK)
          def _():
            fire_gather(i + 2, t)

    plsc.subcore_barrier()
    pltpu.sync_copy(acc.at[pl.ds(s * rps, rps)],
                    out_hbm.at[c, pl.ds(s * rps, rps)])
    if with_count:
      @pl.when(c == 0)
      def _():
        pltpu.sync_copy(cnt_v, cnt_hbm.at[s])

  return seg_sum


_seg_sum_l0 = _make_seg_sum(D0, deep=True, with_count=True)
_seg_sum_l2 = _make_seg_sum(D2, deep=False, with_count=False)


def _stage_a(xp, wl0t, wr0t):
  def body(x_ref, wl_ref, wr_ref, ycat_ref, r0_ref):
    xv = x_ref[...]
    y0 = jnp.dot(xv, wl_ref[...], preferred_element_type=jnp.float32)
    ycat_ref[0] = y0[:, :64]
    ycat_ref[1] = y0[:, 64:]
    r0_ref[...] = jnp.dot(xv, wr_ref[...], preferred_element_type=jnp.float32)

  return pl.pallas_call(
      body,
      grid=(ACC_N // BR,),
      in_specs=[
          pl.BlockSpec((BR, NFEAT), lambda i: (i, 0)),
          pl.BlockSpec((NFEAT, NHID), lambda i: (0, 0)),
          pl.BlockSpec((NFEAT, NHID), lambda i: (0, 0)),
      ],
      out_specs=[
          pl.BlockSpec((NC, BR, D0), lambda i: (0, i, 0)),
          pl.BlockSpec((BR, NHID), lambda i: (i, 0)),
      ],
      out_shape=[
          jax.ShapeDtypeStruct((NC, ACC_N, D0), jnp.float32),
          jax.ShapeDtypeStruct((ACC_N, NHID), jnp.float32),
      ],
  )(xp, wl0t, wr0t)


def _stage_b(p0, cnts, r0, b0, gamma, beta, wl2t, wr2t, b2):
  def body(p_ref, cnt_ref, r0_ref, b0_ref, g_ref, be_ref, wl_ref, wr_ref,
           b2_ref, y2_ref, r2b_ref, inv8_ref):
    agg = jnp.concatenate([p_ref[0], p_ref[1]], axis=1)
    cnt = jnp.sum(cnt_ref[...], axis=1, keepdims=True)
    inv = 1.0 / jnp.maximum(cnt, 1.0)
    pre = agg * inv + r0_ref[...] + b0_ref[...]
    h = jnp.maximum(pre * (g_ref[...] * _BN_SCALE) + be_ref[...], 0.0)
    y2 = jnp.dot(h, wl_ref[...], preferred_element_type=jnp.float32)
    y2_ref[0] = y2[:, :D2]
    y2_ref[1] = y2[:, D2:]
    r2b_ref[...] = jnp.dot(h, wr_ref[...],
                           preferred_element_type=jnp.float32) + b2_ref[...]
    inv8_ref[...] = jnp.broadcast_to(inv, (BR, 8))

  return pl.pallas_call(
      body,
      grid=(ACC_N // BR,),
      in_specs=[
          pl.BlockSpec((NC, BR, D0), lambda i: (0, i, 0)),
          pl.BlockSpec((BR, NS), lambda i: (i, 0)),
          pl.BlockSpec((BR, NHID), lambda i: (i, 0)),
          pl.BlockSpec((NHID,), lambda i: (0,)),
          pl.BlockSpec((NHID,), lambda i: (0,)),
          pl.BlockSpec((NHID,), lambda i: (0,)),
          pl.BlockSpec((NHID, NCLASS), lambda i: (0, 0)),
          pl.BlockSpec((NHID, NCLASS), lambda i: (0, 0)),
          pl.BlockSpec((NCLASS,), lambda i: (0,)),
      ],
      out_specs=[
          pl.BlockSpec((NC, BR, D2), lambda i: (0, i, 0)),
          pl.BlockSpec((BR, NCLASS), lambda i: (i, 0)),
          pl.BlockSpec((BR, 8), lambda i: (i, 0)),
      ],
      out_shape=[
          jax.ShapeDtypeStruct((NC, ACC_N, D2), jnp.float32),
          jax.ShapeDtypeStruct((ACC_N, NCLASS), jnp.float32),
          jax.ShapeDtypeStruct((ACC_N, 8), jnp.float32),
      ],
  )(p0, cnts, r0, b0, gamma, beta, wl2t, wr2t, b2)


def _stage_c(p2, r2b, inv8):
  def body(p_ref, r_ref, inv_ref, out_ref):
    psum = jnp.concatenate([p_ref[0], p_ref[1]], axis=1)
    out_ref[...] = psum * inv_ref[:, 0:1] + r_ref[...]

  return pl.pallas_call(
      body,
      grid=(ACC_N // BR,),
      in_specs=[
          pl.BlockSpec((NC, BR, D2), lambda i: (0, i, 0)),
          pl.BlockSpec((BR, NCLASS), lambda i: (i, 0)),
          pl.BlockSpec((BR, 8), lambda i: (i, 0)),
      ],
      out_specs=pl.BlockSpec((BR, NCLASS), lambda i: (i, 0)),
      out_shape=jax.ShapeDtypeStruct((ACC_N, NCLASS), jnp.float32),
  )(p2, r2b, inv8)


@jax.jit
def kernel(x, edge_index, Wl0, Wr0, b0, gamma, beta, Wl2, Wr2, b2):
  xp = jnp.pad(x, ((0, ACC_N - N), (0, 0)))
  src = jnp.concatenate(
      [edge_index[0], jnp.zeros((E_PAD - E,), jnp.int32)]).reshape(
          NS, NCHUNK, CH)
  src2 = jnp.stack([src, src + ACC_N])
  dst2 = jnp.concatenate(
      [edge_index[1], jnp.full((E_PAD - E,), N, jnp.int32)]).reshape(
          NS, NCHUNK, CH)

  ycat, r0 = _stage_a(xp, Wl0.T, Wr0.T)
  z0 = jnp.zeros((ACC_N // NS, D0), jnp.float32)
  z2 = jnp.zeros((ACC_N // NS, D2), jnp.float32)
  p0, cnts = _seg_sum_l0(ycat.reshape(NC * ACC_N, D0), src2, dst2, z0)
  y2, r2b, inv8 = _stage_b(p0, cnts.T, r0, b0, gamma, beta, Wl2.T, Wr2.T, b2)
  p2 = _seg_sum_l2(y2.reshape(NC * ACC_N, D2), src2, dst2, z2)
  out = _stage_c(p2, r2b, inv8)
  return out[:N]
